# initial kernel scaffold (unmeasured)
import jax
import jax.numpy as jnp
from jax import lax
from jax.experimental import pallas as pl
from jax.experimental.pallas import tpu as pltpu

N_DEV = 4
SCALE = 0.08838834764831843
NEG = -1e30


def kernel(x, Wq, Wo, K_ext, V_ext):
    B, Sq, D = x.shape
    _, Skv, Hq, Dh = K_ext.shape

    x2 = x.reshape(Sq, D)
    k2 = K_ext.reshape(Skv, Hq * Dh)
    v2 = V_ext.reshape(Skv, Hq * Dh)

    def body(x_ref, wq_ref, wo_ref, k_ref, v_ref, out_ref,
             q_buf, acc_buf, ml_buf, ot_buf, sems):
        my = lax.axis_index("i")
        left = (my + N_DEV - 1) % N_DEV
        right = (my + 1) % N_DEV

        barrier = pltpu.get_barrier_semaphore()
        for nbr in (left, right):
            pl.semaphore_signal(barrier, inc=1, device_id=(nbr,),
                                device_id_type=pl.DeviceIdType.MESH)
        pl.semaphore_wait(barrier, 2)

        q2d = lax.dot_general(x_ref[...], wq_ref[...],
                              (((1,), (0,)), ((), ())),
                              preferred_element_type=jnp.float32)
        q_buf[0] = q2d * SCALE
        acc_buf[0] = jnp.zeros((D, Sq), jnp.float32)
        ml_buf[0, 0] = jnp.full((Hq, Sq), NEG, jnp.float32)
        ml_buf[0, 1] = jnp.zeros((Hq, Sq), jnp.float32)

        for r in range(N_DEV):
            for h in range(Hq):
                q_h = q_buf[r, :, h * Dh:(h + 1) * Dh]
                k_h = k_ref[:, h * Dh:(h + 1) * Dh]
                v_h = v_ref[:, h * Dh:(h + 1) * Dh]
                s_t = lax.dot_general(k_h, q_h, (((1,), (1,)), ((), ())),
                                      preferred_element_type=jnp.float32)
                m_prev = ml_buf[r, 0, h, :].reshape(1, Sq)
                l_prev = ml_buf[r, 1, h, :].reshape(1, Sq)
                mj = jnp.max(s_t, axis=0, keepdims=True)
                m_new = jnp.maximum(m_prev, mj)
                alpha = jnp.exp(m_prev - m_new)
                p = jnp.exp(s_t - m_new)
                l_new = l_prev * alpha + jnp.sum(p, axis=0, keepdims=True)
                pv_t = lax.dot_general(v_h, p, (((0,), (0,)), ((), ())),
                                       preferred_element_type=jnp.float32)
                acc_t = acc_buf[r, h * Dh:(h + 1) * Dh, :]
                acc_buf[r, h * Dh:(h + 1) * Dh, :] = acc_t * alpha + pv_t
                ml_buf[r, 0, h, :] = m_new.reshape(Sq)
                ml_buf[r, 1, h, :] = l_new.reshape(Sq)

            nxt = (r + 1) % N_DEV
            copies = []
            if r < N_DEV - 1:
                copies.append(pltpu.make_async_remote_copy(
                    src_ref=q_buf.at[r], dst_ref=q_buf.at[nxt],
                    send_sem=sems.at[0, r], recv_sem=sems.at[3, r],
                    device_id=(right,), device_id_type=pl.DeviceIdType.MESH))
            copies.append(pltpu.make_async_remote_copy(
                src_ref=acc_buf.at[r], dst_ref=acc_buf.at[nxt],
                send_sem=sems.at[1, r], recv_sem=sems.at[4, r],
                device_id=(right,), device_id_type=pl.DeviceIdType.MESH))
            copies.append(pltpu.make_async_remote_copy(
                src_ref=ml_buf.at[r], dst_ref=ml_buf.at[nxt],
                send_sem=sems.at[2, r], recv_sem=sems.at[5, r],
                device_id=(right,), device_id_type=pl.DeviceIdType.MESH))
            for c in copies:
                c.start()
            for c in copies:
                c.wait()

        for h in range(Hq):
            l = ml_buf[0, 1, h, :].reshape(1, Sq)
            ot_buf[h * Dh:(h + 1) * Dh, :] = (
                acc_buf[0, h * Dh:(h + 1) * Dh, :] / l)
        out_ref[...] = lax.dot_general(ot_buf[...], wo_ref[...],
                                       (((0,), (0,)), ((), ())),
                                       preferred_element_type=jnp.float32)

    out2 = pl.pallas_call(
        body,
        out_shape=jax.ShapeDtypeStruct((Sq, D), jnp.float32),
        in_specs=[pl.BlockSpec(memory_space=pltpu.VMEM)] * 5,
        out_specs=pl.BlockSpec(memory_space=pltpu.VMEM),
        scratch_shapes=[
            pltpu.VMEM((N_DEV, Sq, D), jnp.float32),
            pltpu.VMEM((N_DEV, D, Sq), jnp.float32),
            pltpu.VMEM((N_DEV, 2, Hq, Sq), jnp.float32),
            pltpu.VMEM((D, Sq), jnp.float32),
            pltpu.SemaphoreType.DMA((6, N_DEV)),
        ],
        compiler_params=pltpu.CompilerParams(collective_id=0),
    )(x2, Wq, Wo, k2, v2)
    return out2.reshape(B, Sq, D)


# baseline (device time: 210562 ns/iter reference)
import jax
import jax.numpy as jnp
from jax import lax
from jax.experimental import pallas as pl
from jax.experimental.pallas import tpu as pltpu

N_DEV = 4
SCALE = 0.08838834764831843
NEG = -1e30


def kernel(x, Wq, Wo, K_ext, V_ext):
    B, Sq, D = x.shape
    _, Skv, Hq, Dh = K_ext.shape

    x2 = x.reshape(Sq, D)
    k2 = K_ext.reshape(Skv, Hq * Dh)
    v2 = V_ext.reshape(Skv, Hq * Dh)

    def body(x_ref, wq_ref, wo_ref, k_ref, v_ref, out_ref,
             q_buf, acc_buf, ml_buf, ot_buf, sems):
        my = lax.axis_index("i")
        left = (my + N_DEV - 1) % N_DEV
        right = (my + 1) % N_DEV

        barrier = pltpu.get_barrier_semaphore()
        for nbr in (left, right):
            pl.semaphore_signal(barrier, inc=1, device_id=(nbr,),
                                device_id_type=pl.DeviceIdType.MESH)
        pl.semaphore_wait(barrier, 2)

        q2d = lax.dot_general(x_ref[...], wq_ref[...],
                              (((1,), (0,)), ((), ())),
                              preferred_element_type=jnp.float32)
        q_buf[0] = q2d * SCALE
        acc_buf[0] = jnp.zeros((D, Sq), jnp.float32)
        ml_buf[0, 0] = jnp.full((Hq, Sq), NEG, jnp.float32)
        ml_buf[0, 1] = jnp.zeros((Hq, Sq), jnp.float32)

        for r in range(N_DEV):
            for h in range(Hq):
                q_h = q_buf[r, :, h * Dh:(h + 1) * Dh]
                k_h = k_ref[:, h * Dh:(h + 1) * Dh]
                v_h = v_ref[:, h * Dh:(h + 1) * Dh]
                s_t = lax.dot_general(k_h, q_h, (((1,), (1,)), ((), ())),
                                      preferred_element_type=jnp.float32)
                m_prev = ml_buf[r, 0, h, :].reshape(1, Sq)
                l_prev = ml_buf[r, 1, h, :].reshape(1, Sq)
                mj = jnp.max(s_t, axis=0, keepdims=True)
                m_new = jnp.maximum(m_prev, mj)
                alpha = jnp.exp(m_prev - m_new)
                p = jnp.exp(s_t - m_new)
                l_new = l_prev * alpha + jnp.sum(p, axis=0, keepdims=True)
                pv_t = lax.dot_general(v_h, p, (((0,), (0,)), ((), ())),
                                       preferred_element_type=jnp.float32)
                acc_t = acc_buf[r, h * Dh:(h + 1) * Dh, :]
                acc_buf[r, h * Dh:(h + 1) * Dh, :] = acc_t * alpha + pv_t
                ml_buf[r, 0, h, :] = m_new.reshape(Sq)
                ml_buf[r, 1, h, :] = l_new.reshape(Sq)

            nxt = (r + 1) % N_DEV
            copies = []
            if r < N_DEV - 1:
                copies.append(pltpu.make_async_remote_copy(
                    src_ref=q_buf.at[r], dst_ref=q_buf.at[nxt],
                    send_sem=sems.at[0, r], recv_sem=sems.at[3, r],
                    device_id=(right,), device_id_type=pl.DeviceIdType.MESH))
            copies.append(pltpu.make_async_remote_copy(
                src_ref=acc_buf.at[r], dst_ref=acc_buf.at[nxt],
                send_sem=sems.at[1, r], recv_sem=sems.at[4, r],
                device_id=(right,), device_id_type=pl.DeviceIdType.MESH))
            copies.append(pltpu.make_async_remote_copy(
                src_ref=ml_buf.at[r], dst_ref=ml_buf.at[nxt],
                send_sem=sems.at[2, r], recv_sem=sems.at[5, r],
                device_id=(right,), device_id_type=pl.DeviceIdType.MESH))
            for c in copies:
                c.start()
            for c in copies:
                c.wait()

        for h in range(Hq):
            l = ml_buf[0, 1, h, :].reshape(1, Sq)
            ot_buf[h * Dh:(h + 1) * Dh, :] = (
                acc_buf[0, h * Dh:(h + 1) * Dh, :] / l)
        out_ref[...] = lax.dot_general(ot_buf[...], wo_ref[...],
                                       (((0,), (0,)), ((), ())),
                                       preferred_element_type=jnp.float32)

    out2 = pl.pallas_call(
        body,
        out_shape=jax.ShapeDtypeStruct((Sq, D), jnp.float32),
        in_specs=[pl.BlockSpec(memory_space=pltpu.VMEM)] * 5,
        out_specs=pl.BlockSpec(memory_space=pltpu.VMEM),
        scratch_shapes=[
            pltpu.VMEM((N_DEV, Sq, D), jnp.float32),
            pltpu.VMEM((N_DEV, D, Sq), jnp.float32),
            pltpu.VMEM((N_DEV, 2, Hq, Sq), jnp.float32),
            pltpu.VMEM((D, Sq), jnp.float32),
            pltpu.SemaphoreType.DMA((6, N_DEV)),
        ],
        compiler_params=pltpu.CompilerParams(
            collective_id=0,
            vmem_limit_bytes=64 * 1024 * 1024,
        ),
    )(x2, Wq, Wo, k2, v2)
    return out2.reshape(B, Sq, D)


# device time: 144657 ns/iter; 1.4556x vs baseline; 1.4556x over previous
import jax
import jax.numpy as jnp
from jax import lax
from jax.experimental import pallas as pl
from jax.experimental.pallas import tpu as pltpu

N_DEV = 4
SCALE = 0.08838834764831843


def kernel(x, Wq, Wo, K_ext, V_ext):
    B, Sq, D = x.shape
    _, Skv, Hq, Dh = K_ext.shape

    x2 = x.reshape(Sq, D)
    k2 = K_ext.reshape(Skv, Hq * Dh)
    v2 = V_ext.reshape(Skv, Hq * Dh)

    def body(x_ref, wq_ref, wo_ref, k_ref, v_ref, out_ref,
             q_buf, pacc, pml, sacc, sml, racc, rml, sems):
        my = lax.axis_index("i")
        left = (my + N_DEV - 1) % N_DEV
        right = (my + 1) % N_DEV

        barrier = pltpu.get_barrier_semaphore()
        for nbr in (left, right):
            pl.semaphore_signal(barrier, inc=1, device_id=(nbr,),
                                device_id_type=pl.DeviceIdType.MESH)
        pl.semaphore_wait(barrier, 2)

        q2d = lax.dot_general(x_ref[...], wq_ref[...],
                              (((1,), (0,)), ((), ())),
                              preferred_element_type=jnp.float32)
        q_buf[0] = q2d * SCALE

        def q_fwd(d):
            return pltpu.make_async_remote_copy(
                src_ref=q_buf.at[d], dst_ref=q_buf.at[d + 1],
                send_sem=sems.at[0, d], recv_sem=sems.at[1, d],
                device_id=(right,), device_id_type=pl.DeviceIdType.MESH)

        def chunk_copy(s):
            return (
                pltpu.make_async_remote_copy(
                    src_ref=sacc.at[s], dst_ref=racc.at[s],
                    send_sem=sems.at[2, s], recv_sem=sems.at[3, s],
                    device_id=(right,), device_id_type=pl.DeviceIdType.MESH),
                pltpu.make_async_remote_copy(
                    src_ref=sml.at[s], dst_ref=rml.at[s],
                    send_sem=sems.at[4, s], recv_sem=sems.at[5, s],
                    device_id=(right,), device_id_type=pl.DeviceIdType.MESH),
            )

        qf = [q_fwd(d) for d in range(N_DEV - 1)]
        cc = [chunk_copy(s) for s in range(N_DEV - 1)]

        def partial(qslot, acc_ref, ml_ref):
            for h in range(Hq):
                hs = slice(h * Dh, (h + 1) * Dh)
                q_h = q_buf[qslot, :, hs]
                s_t = lax.dot_general(k_ref[:, hs], q_h,
                                      (((1,), (1,)), ((), ())),
                                      preferred_element_type=jnp.float32)
                m_p = jnp.max(s_t, axis=0, keepdims=True)
                p = jnp.exp(s_t - m_p)
                l_p = jnp.sum(p, axis=0, keepdims=True)
                acc_ref[hs, :] = lax.dot_general(
                    v_ref[:, hs], p, (((0,), (0,)), ((), ())),
                    preferred_element_type=jnp.float32)
                ml_ref[0, h] = m_p.reshape(Sq)
                ml_ref[1, h] = l_p.reshape(Sq)

        def merge(s, dst_acc, dst_ml, normalize=False):
            for h in range(Hq):
                hs = slice(h * Dh, (h + 1) * Dh)
                m_in = rml[s, 0, h].reshape(1, Sq)
                l_in = rml[s, 1, h].reshape(1, Sq)
                m_p = pml[0, h].reshape(1, Sq)
                l_p = pml[1, h].reshape(1, Sq)
                m_t = jnp.maximum(m_in, m_p)
                a_in = jnp.exp(m_in - m_t)
                a_p = jnp.exp(m_p - m_t)
                l_t = l_in * a_in + l_p * a_p
                acc_t = racc[s, hs, :] * a_in + pacc[hs, :] * a_p
                if normalize:
                    dst_acc[hs, :] = acc_t / l_t
                else:
                    dst_acc[hs, :] = acc_t
                    dst_ml[0, h] = m_t.reshape(Sq)
                    dst_ml[1, h] = l_t.reshape(Sq)

        qf[0].start()
        qf[0].wait_recv()
        qf[1].start()
        partial(1, sacc.at[0], sml.at[0])
        for c in cc[0]:
            c.start()

        qf[1].wait_recv()
        qf[2].start()
        partial(2, pacc, pml)
        for c in cc[0]:
            c.wait_recv()
        merge(0, sacc.at[1], sml.at[1])
        for c in cc[1]:
            c.start()

        qf[2].wait_recv()
        partial(3, pacc, pml)
        for c in cc[1]:
            c.wait_recv()
        merge(1, sacc.at[2], sml.at[2])
        for c in cc[2]:
            c.start()

        partial(0, pacc, pml)
        for c in cc[2]:
            c.wait_recv()
        merge(2, pacc, pml, normalize=True)

        out_ref[...] = lax.dot_general(pacc[...], wo_ref[...],
                                       (((0,), (0,)), ((), ())),
                                       preferred_element_type=jnp.float32)

        for d in range(N_DEV - 1):
            qf[d].wait_send()
            for c in cc[d]:
                c.wait_send()

    out2 = pl.pallas_call(
        body,
        out_shape=jax.ShapeDtypeStruct((Sq, D), jnp.float32),
        in_specs=[pl.BlockSpec(memory_space=pltpu.VMEM)] * 5,
        out_specs=pl.BlockSpec(memory_space=pltpu.VMEM),
        scratch_shapes=[
            pltpu.VMEM((N_DEV, Sq, D), jnp.float32),
            pltpu.VMEM((D, Sq), jnp.float32),
            pltpu.VMEM((2, Hq, Sq), jnp.float32),
            pltpu.VMEM((N_DEV - 1, D, Sq), jnp.float32),
            pltpu.VMEM((N_DEV - 1, 2, Hq, Sq), jnp.float32),
            pltpu.VMEM((N_DEV - 1, D, Sq), jnp.float32),
            pltpu.VMEM((N_DEV - 1, 2, Hq, Sq), jnp.float32),
            pltpu.SemaphoreType.DMA((6, N_DEV - 1)),
        ],
        compiler_params=pltpu.CompilerParams(
            collective_id=0,
            vmem_limit_bytes=64 * 1024 * 1024,
        ),
    )(x2, Wq, Wo, k2, v2)
    return out2.reshape(B, Sq, D)
